# baseline (speedup 1.0000x reference)
import functools

import jax
import jax.numpy as jnp
from jax import lax
from jax.experimental import pallas as pl
from jax.experimental.pallas import tpu as pltpu
from jax.experimental.pallas import tpu_sc as plsc

_H = 64
_NC = 2
_NS = 16
_NW = _NC * _NS
_CHUNK = 128


def _sc_gather(table, idx3):
    nw, n_chunks, chunk = idx3.shape
    n = nw * n_chunks * chunk
    mesh = plsc.VectorSubcoreMesh(core_axis_name="c", subcore_axis_name="s")

    @functools.partial(
        pl.kernel,
        out_type=jax.ShapeDtypeStruct((n, _H), jnp.float32),
        mesh=mesh,
        compiler_params=pltpu.CompilerParams(use_tc_tiling_on_sc=False),
        scratch_types=[
            pltpu.VMEM((n_chunks, chunk), jnp.int32),
            pltpu.VMEM((chunk, _H), jnp.float32),
            pltpu.SemaphoreType.DMA,
        ],
    )
    def k(table_hbm, idx_hbm, out_hbm, idx_v, rows_v, gsem):
        wid = lax.axis_index("s") * _NC + lax.axis_index("c")
        pltpu.sync_copy(idx_hbm.at[wid], idx_v)

        def body(j, carry):
            pltpu.async_copy(table_hbm.at[idx_v.at[j]], rows_v, gsem).wait()
            base = (wid * n_chunks + j) * chunk
            pltpu.sync_copy(rows_v, out_hbm.at[pl.ds(base, chunk)])
            return carry

        lax.fori_loop(0, n_chunks, body, 0)

    return k(table, idx3)


def _tc_head(h2, w2, b2):
    n2 = h2.shape[0]
    blk = 2048
    grid = (n2 // blk,)

    def body(h_ref, w_ref, b_ref, o_ref):
        s = 1.0 / (1.0 + jnp.exp(-h_ref[...]))
        o_ref[...] = (
            jnp.dot(s, w_ref[...], preferred_element_type=jnp.float32) + b_ref[...]
        )

    return pl.pallas_call(
        body,
        grid=grid,
        in_specs=[
            pl.BlockSpec((blk, 128), lambda i: (i, 0)),
            pl.BlockSpec((128, 4), lambda i: (0, 0)),
            pl.BlockSpec((1, 4), lambda i: (0, 0)),
        ],
        out_specs=pl.BlockSpec((blk, 4), lambda i: (i, 0)),
        out_shape=jax.ShapeDtypeStruct((n2, 4), jnp.float32),
    )(h2, w2, b2)


def kernel(x, table, W, b):
    bsz, fields = x.shape
    n = bsz * fields
    n_chunks = n // (_NW * _CHUNK)
    idx3 = x.reshape(_NW, n_chunks, _CHUNK)

    emb = _sc_gather(table, idx3)

    w2 = jnp.zeros((2 * _H, 4), jnp.float32)
    w2 = w2.at[:_H, :2].set(W.T).at[_H:, 2:].set(W.T)
    b2 = jnp.concatenate([b, b]).reshape(1, 4)

    out4 = _tc_head(emb.reshape(n // 2, 2 * _H), w2, b2)
    return out4.reshape(bsz, fields, 2)


# speedup vs baseline: 1.7280x; 1.7280x over previous
import functools

import jax
import jax.numpy as jnp
from jax import lax
from jax.experimental import pallas as pl
from jax.experimental.pallas import tpu as pltpu
from jax.experimental.pallas import tpu_sc as plsc

_H = 64
_NC = 2
_NS = 16
_NW = _NC * _NS
_CHUNK = 128
_KFIRE = 8


def _tc_head_table(table, W, b):
    v = table.shape[0]
    blk = 8192
    grid = ((v + blk - 1) // blk,)

    def body(t_ref, w_ref, b_ref, o_ref):
        s = 1.0 / (1.0 + jnp.exp(-t_ref[...]))
        y = lax.dot_general(
            w_ref[...], s, (((1,), (1,)), ((), ())),
            preferred_element_type=jnp.float32,
        )
        o_ref[...] = y + b_ref[...]

    return pl.pallas_call(
        body,
        grid=grid,
        in_specs=[
            pl.BlockSpec((blk, _H), lambda i: (i, 0)),
            pl.BlockSpec((2, _H), lambda i: (0, 0)),
            pl.BlockSpec((2, 1), lambda i: (0, 0)),
        ],
        out_specs=pl.BlockSpec((2, blk), lambda i: (0, i)),
        out_shape=jax.ShapeDtypeStruct((2, v), jnp.float32),
    )(table, W, b.reshape(2, 1))


def _sc_lookup(flat, idx_lo, idx_hi):
    nw, n_chunks, chunk = idx_lo.shape
    n = nw * n_chunks * chunk
    n_super = n_chunks // _KFIRE
    sup = _KFIRE * chunk
    mesh = plsc.VectorSubcoreMesh(core_axis_name="c", subcore_axis_name="s")

    @functools.partial(
        pl.kernel,
        out_type=(
            jax.ShapeDtypeStruct((n,), jnp.float32),
            jax.ShapeDtypeStruct((n,), jnp.float32),
        ),
        mesh=mesh,
        compiler_params=pltpu.CompilerParams(use_tc_tiling_on_sc=False),
        scratch_types=[
            pltpu.VMEM((n_chunks, chunk), jnp.int32),
            pltpu.VMEM((n_chunks, chunk), jnp.int32),
            pltpu.VMEM((sup,), jnp.float32),
            pltpu.VMEM((sup,), jnp.float32),
            pltpu.SemaphoreType.DMA,
        ],
    )
    def k(flat_hbm, lo_hbm, hi_hbm, out0_hbm, out1_hbm,
          lo_v, hi_v, buf0_v, buf1_v, gsem):
        wid = lax.axis_index("s") * _NC + lax.axis_index("c")
        pltpu.sync_copy(lo_hbm.at[wid], lo_v)
        pltpu.sync_copy(hi_hbm.at[wid], hi_v)

        def body(sb, carry):
            copies = []
            for bq in range(_KFIRE):
                j = sb * _KFIRE + bq
                copies.append(pltpu.async_copy(
                    flat_hbm.at[lo_v.at[j]],
                    buf0_v.at[pl.ds(bq * chunk, chunk)], gsem))
                copies.append(pltpu.async_copy(
                    flat_hbm.at[hi_v.at[j]],
                    buf1_v.at[pl.ds(bq * chunk, chunk)], gsem))
            for c in copies:
                c.wait()
            base = (wid * n_super + sb) * sup
            pltpu.sync_copy(buf0_v, out0_hbm.at[pl.ds(base, sup)])
            pltpu.sync_copy(buf1_v, out1_hbm.at[pl.ds(base, sup)])
            return carry

        lax.fori_loop(0, n_super, body, 0)

    return k(flat, idx_lo, idx_hi)


def kernel(x, table, W, b):
    bsz, fields = x.shape
    v = table.shape[0]
    n = bsz * fields
    n_chunks = n // (_NW * _CHUNK)
    idx_lo = x.reshape(_NW, n_chunks, _CHUNK)
    idx_hi = idx_lo + v

    planes = _tc_head_table(table, W, b)
    y0, y1 = _sc_lookup(planes.reshape(2 * v), idx_lo, idx_hi)

    out = jnp.stack([y0, y1], axis=-1)
    return out.reshape(bsz, fields, 2)
